# Initial kernel scaffold; baseline (speedup 1.0000x reference)
import jax
import jax.numpy as jnp
from jax.experimental import pallas as pl


def kernel(tokens, lengths, queries, W1a, b1a, W1b, b1b, W2a, b2a, W2b, b2b):
    raise NotImplementedError("write your pallas kernel here")


# speedup vs baseline: 3.2809x; 3.2809x over previous
import functools

import jax
import jax.numpy as jnp
from jax.experimental import pallas as pl
from jax.experimental.pallas import tpu as pltpu

B, S, D, M = 16, 4096, 768, 4
CS = 512
NC = (S + 1 + CS - 1) // CS
QP = 8
SCALE = D ** -0.5


def _pool_attn_body(lens_ref, tok_ref, q_ref, out1_ref, out2_ref,
                    sum_s, max_s, min_s, acc_s, l_s):
    b = pl.program_id(0)
    c = pl.program_id(1)
    L = lens_ref[b]

    @pl.when(c == 0)
    def _init():
        sum_s[...] = jnp.zeros_like(sum_s)
        max_s[...] = jnp.full_like(max_s, -1e30)
        min_s[...] = jnp.full_like(min_s, 1e30)
        acc_s[...] = jnp.zeros_like(acc_s)
        l_s[...] = jnp.zeros_like(l_s)

    @pl.when(c * CS <= L)
    def _accum():
        x = tok_ref[0]
        r = jax.lax.broadcasted_iota(jnp.int32, (CS, 1), 0) + c * CS
        valid = (r >= 1) & (r <= L)
        vf = valid.astype(x.dtype)
        sum_s[0:1, :] += jnp.sum(x * vf, axis=0, keepdims=True)
        max_s[0:1, :] = jnp.maximum(
            max_s[0:1, :],
            jnp.max(jnp.where(valid, x, -1e30), axis=0, keepdims=True))
        min_s[0:1, :] = jnp.minimum(
            min_s[0:1, :],
            jnp.min(jnp.where(valid, x, 1e30), axis=0, keepdims=True))
        s = jax.lax.dot_general(x, q_ref[...], (((1,), (1,)), ((), ())),
                                preferred_element_type=jnp.float32) * SCALE
        p = jnp.where(valid, jnp.exp(s), 0.0)
        l_s[...] += jnp.sum(p, axis=0)[:, None]
        acc_s[...] += jax.lax.dot_general(p, x, (((0,), (0,)), ((), ())),
                                          preferred_element_type=jnp.float32)

    @pl.when(c == NC - 1)
    def _finalize():
        inv_len = 1.0 / L.astype(jnp.float32)
        out1_ref[0, 0:1, 0:D] = sum_s[0:1, :] * inv_len
        out1_ref[0, 0:1, D:2 * D] = max_s[0:1, :]
        out1_ref[0, 0:1, 2 * D:3 * D] = min_s[0:1, :]
        out2_ref[0] = acc_s[...] / l_s[:, 0:1]


def _pool_attn(lengths_i32, tokens, qpad):
    grid_spec = pltpu.PrefetchScalarGridSpec(
        num_scalar_prefetch=1,
        grid=(B, NC),
        in_specs=[
            pl.BlockSpec((1, CS, D),
                         lambda b, c, lens: (b, jnp.minimum(c, lens[b] // CS), 0)),
            pl.BlockSpec((QP, D), lambda b, c, lens: (0, 0)),
        ],
        out_specs=[
            pl.BlockSpec((1, 1, 3 * D), lambda b, c, lens: (b, 0, 0)),
            pl.BlockSpec((1, QP, D), lambda b, c, lens: (b, 0, 0)),
        ],
        scratch_shapes=[
            pltpu.VMEM((8, D), jnp.float32),
            pltpu.VMEM((8, D), jnp.float32),
            pltpu.VMEM((8, D), jnp.float32),
            pltpu.VMEM((QP, D), jnp.float32),
            pltpu.VMEM((QP, 128), jnp.float32),
        ],
    )
    return pl.pallas_call(
        _pool_attn_body,
        grid_spec=grid_spec,
        out_shape=[
            jax.ShapeDtypeStruct((B, 1, 3 * D), jnp.float32),
            jax.ShapeDtypeStruct((B, QP, D), jnp.float32),
        ],
    )(lengths_i32, tokens, qpad)


def _gelu_exact(x):
    return 0.5 * x * (1.0 + jax.lax.erf(x * (2.0 ** -0.5)))


def _mlp_body(x1_ref, x2_ref, W1a_ref, b1a_ref, W1b_ref, b1b_ref,
              W2a_ref, b2a_ref, W2b_ref, b2b_ref, out_ref):
    h1 = _gelu_exact(
        jnp.dot(x1_ref[...], W1a_ref[...], preferred_element_type=jnp.float32)
        + b1a_ref[...])
    o1 = jnp.dot(h1, W1b_ref[...], preferred_element_type=jnp.float32) + b1b_ref[...]
    h2 = _gelu_exact(
        jnp.dot(x2_ref[...], W2a_ref[...], preferred_element_type=jnp.float32)
        + b2a_ref[...])
    o2 = jnp.dot(h2, W2b_ref[...], preferred_element_type=jnp.float32) + b2b_ref[...]
    out_ref[:, 0:D] = o1
    out_ref[:, D:2 * D] = o2


def _mlp(x1, x2, W1a, b1a, W1b, b1b, W2a, b2a, W2b, b2b):
    return pl.pallas_call(
        _mlp_body,
        out_shape=jax.ShapeDtypeStruct((B, 2 * D), jnp.float32),
    )(x1, x2, W1a, b1a, W1b, b1b, W2a, b2a, W2b, b2b)


@jax.jit
def kernel(tokens, lengths, queries, W1a, b1a, W1b, b1b, W2a, b2a, W2b, b2b):
    lengths_i32 = lengths.astype(jnp.int32)
    qpad = jnp.zeros((QP, D), jnp.float32).at[:M].set(queries)
    pooled_trad, pmp8 = _pool_attn(lengths_i32, tokens, qpad)
    pooled_trad = pooled_trad.reshape(B, 3 * D)
    clf = tokens[:, 0]
    x2 = jnp.concatenate([pmp8[:, :M, :].reshape(B, M * D), clf], axis=-1)
    return _mlp(pooled_trad, x2,
                W1a, b1a.reshape(1, D), W1b, b1b.reshape(1, D),
                W2a, b2a.reshape(1, D), W2b, b2b.reshape(1, D))


# speedup vs baseline: 3.2953x; 1.0044x over previous
import functools

import jax
import jax.numpy as jnp
from jax.experimental import pallas as pl
from jax.experimental.pallas import tpu as pltpu

B, S, D, M = 16, 4096, 768, 4
CS = 512
NC = (S + 1 + CS - 1) // CS
QP = 8
SCALE = D ** -0.5


def _pool_attn_body(lens_ref, tok_ref, q_ref, out1_ref, out2_ref,
                    max_s, min_s, acc_s, l_s):
    b = pl.program_id(0)
    c = pl.program_id(1)
    L = lens_ref[b]

    @pl.when(c == 0)
    def _init():
        max_s[...] = jnp.full_like(max_s, -1e30)
        min_s[...] = jnp.full_like(min_s, 1e30)
        acc_s[...] = jnp.zeros_like(acc_s)
        l_s[...] = jnp.zeros_like(l_s)

    full = (c > 0) & (c * CS + (CS - 1) <= L)

    @pl.when(full)
    def _accum_full():
        x = tok_ref[0]
        max_s[0:1, :] = jnp.maximum(max_s[0:1, :],
                                    jnp.max(x, axis=0, keepdims=True))
        min_s[0:1, :] = jnp.minimum(min_s[0:1, :],
                                    jnp.min(x, axis=0, keepdims=True))
        s = jax.lax.dot_general(x, q_ref[...], (((1,), (1,)), ((), ())),
                                preferred_element_type=jnp.float32) * SCALE
        p = jnp.exp(s)
        l_s[...] += jnp.sum(p, axis=0)[:, None]
        acc_s[...] += jax.lax.dot_general(p, x, (((0,), (0,)), ((), ())),
                                          preferred_element_type=jnp.float32)

    @pl.when((c * CS <= L) & jnp.logical_not(full))
    def _accum_boundary():
        x = tok_ref[0]
        r = jax.lax.broadcasted_iota(jnp.int32, (CS, 1), 0) + c * CS
        valid = (r >= 1) & (r <= L)
        max_s[0:1, :] = jnp.maximum(
            max_s[0:1, :],
            jnp.max(jnp.where(valid, x, -1e30), axis=0, keepdims=True))
        min_s[0:1, :] = jnp.minimum(
            min_s[0:1, :],
            jnp.min(jnp.where(valid, x, 1e30), axis=0, keepdims=True))
        s = jax.lax.dot_general(x, q_ref[...], (((1,), (1,)), ((), ())),
                                preferred_element_type=jnp.float32) * SCALE
        p = jnp.where(valid, jnp.exp(s), 0.0)
        l_s[...] += jnp.sum(p, axis=0)[:, None]
        acc_s[...] += jax.lax.dot_general(p, x, (((0,), (0,)), ((), ())),
                                          preferred_element_type=jnp.float32)

    @pl.when(c == NC - 1)
    def _finalize():
        inv_len = 1.0 / L.astype(jnp.float32)
        out1_ref[0, 0:1, 0:D] = acc_s[M:M + 1, :] * inv_len
        out1_ref[0, 0:1, D:2 * D] = max_s[0:1, :]
        out1_ref[0, 0:1, 2 * D:3 * D] = min_s[0:1, :]
        out2_ref[0] = acc_s[...] / l_s[:, 0:1]


def _pool_attn(lengths_i32, tokens, qpad):
    grid_spec = pltpu.PrefetchScalarGridSpec(
        num_scalar_prefetch=1,
        grid=(B, NC),
        in_specs=[
            pl.BlockSpec((1, CS, D),
                         lambda b, c, lens: (b, jnp.minimum(c, lens[b] // CS), 0)),
            pl.BlockSpec((QP, D), lambda b, c, lens: (0, 0)),
        ],
        out_specs=[
            pl.BlockSpec((1, 1, 3 * D), lambda b, c, lens: (b, 0, 0)),
            pl.BlockSpec((1, QP, D), lambda b, c, lens: (b, 0, 0)),
        ],
        scratch_shapes=[
            pltpu.VMEM((8, D), jnp.float32),
            pltpu.VMEM((8, D), jnp.float32),
            pltpu.VMEM((QP, D), jnp.float32),
            pltpu.VMEM((QP, 128), jnp.float32),
        ],
    )
    return pl.pallas_call(
        _pool_attn_body,
        grid_spec=grid_spec,
        out_shape=[
            jax.ShapeDtypeStruct((B, 1, 3 * D), jnp.float32),
            jax.ShapeDtypeStruct((B, QP, D), jnp.float32),
        ],
    )(lengths_i32, tokens, qpad)


def _gelu_exact(x):
    return 0.5 * x * (1.0 + jax.lax.erf(x * (2.0 ** -0.5)))


def _mlp_body(x1_ref, x2_ref, W1a_ref, b1a_ref, W1b_ref, b1b_ref,
              W2a_ref, b2a_ref, W2b_ref, b2b_ref, out_ref):
    h1 = _gelu_exact(
        jnp.dot(x1_ref[...], W1a_ref[...], preferred_element_type=jnp.float32)
        + b1a_ref[...])
    o1 = jnp.dot(h1, W1b_ref[...], preferred_element_type=jnp.float32) + b1b_ref[...]
    h2 = _gelu_exact(
        jnp.dot(x2_ref[...], W2a_ref[...], preferred_element_type=jnp.float32)
        + b2a_ref[...])
    o2 = jnp.dot(h2, W2b_ref[...], preferred_element_type=jnp.float32) + b2b_ref[...]
    out_ref[:, 0:D] = o1
    out_ref[:, D:2 * D] = o2


def _mlp(x1, x2, W1a, b1a, W1b, b1b, W2a, b2a, W2b, b2b):
    return pl.pallas_call(
        _mlp_body,
        out_shape=jax.ShapeDtypeStruct((B, 2 * D), jnp.float32),
    )(x1, x2, W1a, b1a, W1b, b1b, W2a, b2a, W2b, b2b)


@jax.jit
def kernel(tokens, lengths, queries, W1a, b1a, W1b, b1b, W2a, b2a, W2b, b2b):
    lengths_i32 = lengths.astype(jnp.int32)
    qpad = jnp.zeros((QP, D), jnp.float32).at[:M].set(queries)
    pooled_trad, pmp8 = _pool_attn(lengths_i32, tokens, qpad)
    pooled_trad = pooled_trad.reshape(B, 3 * D)
    clf = tokens[:, 0]
    x2 = jnp.concatenate([pmp8[:, :M, :].reshape(B, M * D), clf], axis=-1)
    return _mlp(pooled_trad, x2,
                W1a, b1a.reshape(1, D), W1b, b1b.reshape(1, D),
                W2a, b2a.reshape(1, D), W2b, b2b.reshape(1, D))


# speedup vs baseline: 3.3932x; 1.0297x over previous
import functools

import jax
import jax.numpy as jnp
from jax.experimental import pallas as pl
from jax.experimental.pallas import tpu as pltpu

B, S, D, M = 16, 4096, 768, 4
CS = 1024
NC = (S + 1 + CS - 1) // CS
QP = 8
SCALE = D ** -0.5


def _pool_attn_body(lens_ref, tok_ref, q_ref, out1_ref, out2_ref,
                    max_s, min_s, acc_s, l_s):
    b = pl.program_id(0)
    c = pl.program_id(1)
    L = lens_ref[b]

    @pl.when(c == 0)
    def _init():
        max_s[...] = jnp.full_like(max_s, -1e30)
        min_s[...] = jnp.full_like(min_s, 1e30)
        acc_s[...] = jnp.zeros_like(acc_s)
        l_s[...] = jnp.zeros_like(l_s)

    full = (c > 0) & (c * CS + (CS - 1) <= L)

    @pl.when(full)
    def _accum_full():
        x = tok_ref[0]
        max_s[0:1, :] = jnp.maximum(max_s[0:1, :],
                                    jnp.max(x, axis=0, keepdims=True))
        min_s[0:1, :] = jnp.minimum(min_s[0:1, :],
                                    jnp.min(x, axis=0, keepdims=True))
        s = jax.lax.dot_general(x, q_ref[...], (((1,), (1,)), ((), ())),
                                preferred_element_type=jnp.float32) * SCALE
        p = jnp.exp(s)
        l_s[...] += jnp.sum(p, axis=0)[:, None]
        acc_s[...] += jax.lax.dot_general(p, x, (((0,), (0,)), ((), ())),
                                          preferred_element_type=jnp.float32)

    @pl.when((c * CS <= L) & jnp.logical_not(full))
    def _accum_boundary():
        x = tok_ref[0]
        r = jax.lax.broadcasted_iota(jnp.int32, (CS, 1), 0) + c * CS
        valid = (r >= 1) & (r <= L)
        max_s[0:1, :] = jnp.maximum(
            max_s[0:1, :],
            jnp.max(jnp.where(valid, x, -1e30), axis=0, keepdims=True))
        min_s[0:1, :] = jnp.minimum(
            min_s[0:1, :],
            jnp.min(jnp.where(valid, x, 1e30), axis=0, keepdims=True))
        s = jax.lax.dot_general(x, q_ref[...], (((1,), (1,)), ((), ())),
                                preferred_element_type=jnp.float32) * SCALE
        p = jnp.where(valid, jnp.exp(s), 0.0)
        l_s[...] += jnp.sum(p, axis=0)[:, None]
        acc_s[...] += jax.lax.dot_general(p, x, (((0,), (0,)), ((), ())),
                                          preferred_element_type=jnp.float32)

    @pl.when(c == NC - 1)
    def _finalize():
        inv_len = 1.0 / L.astype(jnp.float32)
        out1_ref[0, 0:1, 0:D] = acc_s[M:M + 1, :] * inv_len
        out1_ref[0, 0:1, D:2 * D] = max_s[0:1, :]
        out1_ref[0, 0:1, 2 * D:3 * D] = min_s[0:1, :]
        out2_ref[0] = acc_s[...] / l_s[:, 0:1]


def _pool_attn(lengths_i32, tokens, qpad):
    grid_spec = pltpu.PrefetchScalarGridSpec(
        num_scalar_prefetch=1,
        grid=(B, NC),
        in_specs=[
            pl.BlockSpec((1, CS, D),
                         lambda b, c, lens: (b, jnp.minimum(c, lens[b] // CS), 0)),
            pl.BlockSpec((QP, D), lambda b, c, lens: (0, 0)),
        ],
        out_specs=[
            pl.BlockSpec((1, 1, 3 * D), lambda b, c, lens: (b, 0, 0)),
            pl.BlockSpec((1, QP, D), lambda b, c, lens: (b, 0, 0)),
        ],
        scratch_shapes=[
            pltpu.VMEM((8, D), jnp.float32),
            pltpu.VMEM((8, D), jnp.float32),
            pltpu.VMEM((QP, D), jnp.float32),
            pltpu.VMEM((QP, 128), jnp.float32),
        ],
    )
    return pl.pallas_call(
        _pool_attn_body,
        grid_spec=grid_spec,
        out_shape=[
            jax.ShapeDtypeStruct((B, 1, 3 * D), jnp.float32),
            jax.ShapeDtypeStruct((B, QP, D), jnp.float32),
        ],
    )(lengths_i32, tokens, qpad)


def _gelu_exact(x):
    return 0.5 * x * (1.0 + jax.lax.erf(x * (2.0 ** -0.5)))


def _mlp_body(x1_ref, x2_ref, W1a_ref, b1a_ref, W1b_ref, b1b_ref,
              W2a_ref, b2a_ref, W2b_ref, b2b_ref, out_ref):
    h1 = _gelu_exact(
        jnp.dot(x1_ref[...], W1a_ref[...], preferred_element_type=jnp.float32)
        + b1a_ref[...])
    o1 = jnp.dot(h1, W1b_ref[...], preferred_element_type=jnp.float32) + b1b_ref[...]
    h2 = _gelu_exact(
        jnp.dot(x2_ref[...], W2a_ref[...], preferred_element_type=jnp.float32)
        + b2a_ref[...])
    o2 = jnp.dot(h2, W2b_ref[...], preferred_element_type=jnp.float32) + b2b_ref[...]
    out_ref[:, 0:D] = o1
    out_ref[:, D:2 * D] = o2


def _mlp(x1, x2, W1a, b1a, W1b, b1b, W2a, b2a, W2b, b2b):
    return pl.pallas_call(
        _mlp_body,
        out_shape=jax.ShapeDtypeStruct((B, 2 * D), jnp.float32),
    )(x1, x2, W1a, b1a, W1b, b1b, W2a, b2a, W2b, b2b)


@jax.jit
def kernel(tokens, lengths, queries, W1a, b1a, W1b, b1b, W2a, b2a, W2b, b2b):
    lengths_i32 = lengths.astype(jnp.int32)
    qpad = jnp.zeros((QP, D), jnp.float32).at[:M].set(queries)
    pooled_trad, pmp8 = _pool_attn(lengths_i32, tokens, qpad)
    pooled_trad = pooled_trad.reshape(B, 3 * D)
    clf = tokens[:, 0]
    x2 = jnp.concatenate([pmp8[:, :M, :].reshape(B, M * D), clf], axis=-1)
    return _mlp(pooled_trad, x2,
                W1a, b1a.reshape(1, D), W1b, b1b.reshape(1, D),
                W2a, b2a.reshape(1, D), W2b, b2b.reshape(1, D))
